# baseline (speedup 1.0000x reference)
import jax
import jax.numpy as jnp
from jax import lax
from jax.experimental import pallas as pl
from jax.experimental.pallas import tpu as pltpu
from jax.experimental.pallas import tpu_sc as plsc

SEQ = 4096
EMB = 128
BATCH = 4
NUM_INPUTS = 2
NUM_CORES = 2
NUM_SUBCORES = 16
NW = NUM_CORES * NUM_SUBCORES
WROWS = SEQ // NW
CH = 32
NCHUNK = NUM_INPUTS * WROWS // CH
LANES = 16
GROUPS = EMB // LANES


def _body(x0, x1, upw, lpw, rtw, out,
          pb, xb0, xb1, ob0, ob1, rv, lp_v, rot_v,
          s_in0, s_in1, s_out0, s_out1, s_small):
    cid = lax.axis_index("c")
    sid = lax.axis_index("s")
    w = sid * NUM_CORES + cid
    s0 = pl.multiple_of(w * WROWS, WROWS)

    d_lp = pltpu.async_copy(lpw, lp_v, s_small)
    d_rt = pltpu.async_copy(rtw, rot_v, s_small)
    d_pb = pltpu.async_copy(upw.at[pl.ds(s0, WROWS)], pb, s_small)

    def start_in(k, xb, sem):
        @pl.when(k <= 3)
        def _():
            ib = pl.multiple_of(s0 + k * CH, CH)
            pltpu.async_copy(x0.at[:, pl.ds(ib, CH)], xb, sem)

        @pl.when(k >= 4)
        def _():
            ib = pl.multiple_of(s0 + (k - 4) * CH, CH)
            pltpu.async_copy(x1.at[:, pl.ds(ib, CH)], xb, sem)

    def wait_in(xb, sem):
        pltpu.make_async_copy(x0.at[:, pl.ds(0, CH)], xb, sem).wait()

    def wait_out(ob, sem):
        pltpu.make_async_copy(ob, out.at[pl.ds(0, CH)], sem).wait()

    start_in(jnp.int32(0), xb0, s_in0)
    start_in(jnp.int32(1), xb1, s_in1)

    d_lp.wait()
    d_rt.wait()
    d_pb.wait()

    lp0 = [lp_v[0, pl.ds(g * LANES, LANES)] for g in range(GROUPS)]
    lp1 = [lp_v[1, pl.ds(g * LANES, LANES)] for g in range(GROUPS)]

    def compute(k, xb, ob):
        m = jnp.where(k <= 3, jnp.float32(1.0), jnp.float32(0.0))
        lpk = [lp1[g] + (lp0[g] - lp1[g]) * m for g in range(GROUPS)]
        poff = (k - 4 * jnp.where(k >= 4, 1, 0)) * CH

        @plsc.parallel_loop(0, CH, unroll=2)
        def _(r):
            pi = poff + r
            for g in range(GROUPS):
                col = pl.ds(g * LANES, LANES)
                pv = pb[pi, col] + lpk[g]
                for b in range(BATCH):
                    ob[r, b, col] = xb[b, r, col] + pv

    def start_out(k, ob, sem):
        obase = 1 + s0 + k * CH + jnp.where(k >= 4, SEQ - 4 * CH, 0)
        pltpu.async_copy(ob, out.at[pl.ds(obase, CH)], sem)

    @pl.when(w == 0)
    def _():
        for b in range(BATCH):
            for g in range(GROUPS):
                col = pl.ds(g * LANES, LANES)
                rv[0, b, col] = rot_v[0, col]
        pltpu.async_copy(rv, out.at[pl.ds(0, 1)], s_small)
        pltpu.make_async_copy(rv, out.at[pl.ds(0, 1)], s_small).wait()

    wait_in(xb0, s_in0)
    compute(jnp.int32(0), xb0, ob0)
    start_out(jnp.int32(0), ob0, s_out0)

    start_in(jnp.int32(2), xb0, s_in0)
    wait_in(xb1, s_in1)
    compute(jnp.int32(1), xb1, ob1)
    start_out(jnp.int32(1), ob1, s_out1)

    def round_body(t, acc):
        k1 = 2 * t + 2

        start_in(k1 + 1, xb1, s_in1)
        wait_in(xb0, s_in0)
        wait_out(ob0, s_out0)
        compute(k1, xb0, ob0)
        start_out(k1, ob0, s_out0)

        @pl.when(k1 + 2 < NCHUNK)
        def _():
            start_in(k1 + 2, xb0, s_in0)

        wait_in(xb1, s_in1)
        wait_out(ob1, s_out1)
        compute(k1 + 1, xb1, ob1)
        start_out(k1 + 1, ob1, s_out1)
        return acc

    lax.fori_loop(0, NCHUNK // 2 - 1, round_body, 0)

    wait_out(ob0, s_out0)
    wait_out(ob1, s_out1)


def kernel(x0, x1, unique_pos_w, layer_pos_w, rot_token_w):
    mesh = plsc.VectorSubcoreMesh(core_axis_name="c", subcore_axis_name="s")
    f32 = jnp.float32
    run = pl.kernel(
        _body,
        out_type=jax.ShapeDtypeStruct((NUM_INPUTS * SEQ + 1, BATCH, EMB), f32),
        mesh=mesh,
        scratch_types=[
            pltpu.VMEM((WROWS, EMB), f32),
            pltpu.VMEM((BATCH, CH, EMB), f32),
            pltpu.VMEM((BATCH, CH, EMB), f32),
            pltpu.VMEM((CH, BATCH, EMB), f32),
            pltpu.VMEM((CH, BATCH, EMB), f32),
            pltpu.VMEM((1, BATCH, EMB), f32),
            pltpu.VMEM((NUM_INPUTS, EMB), f32),
            pltpu.VMEM((1, EMB), f32),
            pltpu.SemaphoreType.DMA,
            pltpu.SemaphoreType.DMA,
            pltpu.SemaphoreType.DMA,
            pltpu.SemaphoreType.DMA,
            pltpu.SemaphoreType.DMA,
        ],
    )
    p = run(x0, x1, unique_pos_w, layer_pos_w, rot_token_w)
    return jnp.transpose(p, (1, 0, 2))


# speedup vs baseline: 1.0016x; 1.0016x over previous
import jax
import jax.numpy as jnp
from jax import lax
from jax.experimental import pallas as pl
from jax.experimental.pallas import tpu as pltpu
from jax.experimental.pallas import tpu_sc as plsc

SEQ = 4096
EMB = 128
BATCH = 4
NUM_INPUTS = 2
NUM_CORES = 2
NUM_SUBCORES = 16
NW = NUM_CORES * NUM_SUBCORES
WROWS = SEQ // NW
CH = 32
NCHUNK = NUM_INPUTS * WROWS // CH
LANES = 16
GROUPS = EMB // LANES


def _body(x0, x1, upw, lpw, rtw, out,
          pb, xb0, xb1, xb2, ob0, ob1, ob2, rv, lp_v, rot_v,
          s_in0, s_in1, s_in2, s_out0, s_out1, s_out2, s_small):
    cid = lax.axis_index("c")
    sid = lax.axis_index("s")
    w = sid * NUM_CORES + cid
    s0 = pl.multiple_of(w * WROWS, WROWS)

    d_lp = pltpu.async_copy(lpw, lp_v, s_small)
    d_rt = pltpu.async_copy(rtw, rot_v, s_small)
    d_pb = pltpu.async_copy(upw.at[pl.ds(s0, WROWS)], pb, s_small)

    def start_in(k, xb, sem):
        @pl.when(k <= 3)
        def _():
            ib = pl.multiple_of(s0 + k * CH, CH)
            pltpu.async_copy(x0.at[:, pl.ds(ib, CH)], xb, sem)

        @pl.when(k >= 4)
        def _():
            ib = pl.multiple_of(s0 + (k - 4) * CH, CH)
            pltpu.async_copy(x1.at[:, pl.ds(ib, CH)], xb, sem)

    def wait_in(xb, sem):
        pltpu.make_async_copy(x0.at[:, pl.ds(0, CH)], xb, sem).wait()

    def wait_out(ob, sem):
        pltpu.make_async_copy(ob, out.at[pl.ds(0, CH)], sem).wait()

    start_in(jnp.int32(0), xb0, s_in0)
    start_in(jnp.int32(1), xb1, s_in1)
    start_in(jnp.int32(2), xb2, s_in2)

    d_lp.wait()
    d_rt.wait()
    d_pb.wait()

    lp0 = [lp_v[0, pl.ds(g * LANES, LANES)] for g in range(GROUPS)]
    lp1 = [lp_v[1, pl.ds(g * LANES, LANES)] for g in range(GROUPS)]

    def compute(k, xb, ob):
        m = jnp.where(k <= 3, jnp.float32(1.0), jnp.float32(0.0))
        lpk = [lp1[g] + (lp0[g] - lp1[g]) * m for g in range(GROUPS)]
        poff = (k - 4 * jnp.where(k >= 4, 1, 0)) * CH

        @plsc.parallel_loop(0, CH, unroll=1)
        def _(r):
            pi = poff + r
            for g in range(GROUPS):
                col = pl.ds(g * LANES, LANES)
                pv = pb[pi, col] + lpk[g]
                for b in range(BATCH):
                    ob[r, b, col] = xb[b, r, col] + pv

    def start_out(k, ob, sem):
        obase = 1 + s0 + k * CH + jnp.where(k >= 4, SEQ - 4 * CH, 0)
        pltpu.async_copy(ob, out.at[pl.ds(obase, CH)], sem)

    @pl.when(w == 0)
    def _():
        for b in range(BATCH):
            for g in range(GROUPS):
                col = pl.ds(g * LANES, LANES)
                rv[0, b, col] = rot_v[0, col]
        pltpu.async_copy(rv, out.at[pl.ds(0, 1)], s_small)
        pltpu.make_async_copy(rv, out.at[pl.ds(0, 1)], s_small).wait()

    wait_in(xb0, s_in0)
    compute(jnp.int32(0), xb0, ob0)
    start_out(jnp.int32(0), ob0, s_out0)

    start_in(jnp.int32(3), xb0, s_in0)
    wait_in(xb1, s_in1)
    compute(jnp.int32(1), xb1, ob1)
    start_out(jnp.int32(1), ob1, s_out1)

    def round_body(t, acc):
        ka = 3 * t + 2

        start_in(ka + 2, xb1, s_in1)

        wait_in(xb2, s_in2)
        @pl.when(t >= 1)
        def _():
            wait_out(ob2, s_out2)
        compute(ka, xb2, ob2)
        start_out(ka, ob2, s_out2)

        @pl.when(t == 0)
        def _():
            start_in(ka + 3, xb2, s_in2)

        wait_in(xb0, s_in0)
        wait_out(ob0, s_out0)
        compute(ka + 1, xb0, ob0)
        start_out(ka + 1, ob0, s_out0)

        @pl.when(t == 0)
        def _():
            start_in(ka + 4, xb0, s_in0)

        wait_in(xb1, s_in1)
        wait_out(ob1, s_out1)
        compute(ka + 2, xb1, ob1)
        start_out(ka + 2, ob1, s_out1)
        return acc

    lax.fori_loop(0, 2, round_body, 0)

    wait_out(ob2, s_out2)
    wait_out(ob0, s_out0)
    wait_out(ob1, s_out1)


def kernel(x0, x1, unique_pos_w, layer_pos_w, rot_token_w):
    mesh = plsc.VectorSubcoreMesh(core_axis_name="c", subcore_axis_name="s")
    f32 = jnp.float32
    run = pl.kernel(
        _body,
        out_type=jax.ShapeDtypeStruct((NUM_INPUTS * SEQ + 1, BATCH, EMB), f32),
        mesh=mesh,
        scratch_types=[
            pltpu.VMEM((WROWS, EMB), f32),
            pltpu.VMEM((BATCH, CH, EMB), f32),
            pltpu.VMEM((BATCH, CH, EMB), f32),
            pltpu.VMEM((BATCH, CH, EMB), f32),
            pltpu.VMEM((CH, BATCH, EMB), f32),
            pltpu.VMEM((CH, BATCH, EMB), f32),
            pltpu.VMEM((CH, BATCH, EMB), f32),
            pltpu.VMEM((1, BATCH, EMB), f32),
            pltpu.VMEM((NUM_INPUTS, EMB), f32),
            pltpu.VMEM((1, EMB), f32),
            pltpu.SemaphoreType.DMA,
            pltpu.SemaphoreType.DMA,
            pltpu.SemaphoreType.DMA,
            pltpu.SemaphoreType.DMA,
            pltpu.SemaphoreType.DMA,
            pltpu.SemaphoreType.DMA,
            pltpu.SemaphoreType.DMA,
        ],
    )
    p = run(x0, x1, unique_pos_w, layer_pos_w, rot_token_w)
    return jnp.transpose(p, (1, 0, 2))


# speedup vs baseline: 1.0121x; 1.0105x over previous
import jax
import jax.numpy as jnp
from jax import lax
from jax.experimental import pallas as pl
from jax.experimental.pallas import tpu as pltpu
from jax.experimental.pallas import tpu_sc as plsc

SEQ = 4096
EMB = 128
BATCH = 4
NUM_INPUTS = 2
NUM_CORES = 2
NUM_SUBCORES = 16
NW = NUM_CORES * NUM_SUBCORES
WROWS = SEQ // NW
CH = 32
NCHUNK = NUM_INPUTS * WROWS // CH
LANES = 16
GROUPS = EMB // LANES


def _body(x0, x1, upw, lpw, rtw, out,
          pb, xb0, xb1, ob0, ob1, rv, lp_v, rot_v,
          s_in0, s_in1, s_out0, s_out1, s_small):
    cid = lax.axis_index("c")
    sid = lax.axis_index("s")
    w = sid * NUM_CORES + cid
    s0 = pl.multiple_of(w * WROWS, WROWS)

    d_lp = pltpu.async_copy(lpw, lp_v, s_small)
    d_rt = pltpu.async_copy(rtw, rot_v, s_small)
    d_pb = pltpu.async_copy(upw.at[pl.ds(s0, WROWS)], pb, s_small)

    def start_in(k, xb, sem):
        @pl.when(k <= 3)
        def _():
            ib = pl.multiple_of(s0 + k * CH, CH)
            pltpu.async_copy(x0.at[:, pl.ds(ib, CH)], xb, sem)

        @pl.when(k >= 4)
        def _():
            ib = pl.multiple_of(s0 + (k - 4) * CH, CH)
            pltpu.async_copy(x1.at[:, pl.ds(ib, CH)], xb, sem)

    def wait_in(xb, sem):
        pltpu.make_async_copy(x0.at[:, pl.ds(0, CH)], xb, sem).wait()

    def wait_out(ob, sem):
        pltpu.make_async_copy(ob, out.at[pl.ds(0, CH)], sem).wait()

    start_in(jnp.int32(0), xb0, s_in0)
    start_in(jnp.int32(1), xb1, s_in1)

    d_lp.wait()
    d_rt.wait()
    d_pb.wait()

    lp0 = [lp_v[0, pl.ds(g * LANES, LANES)] for g in range(GROUPS)]
    lp1 = [lp_v[1, pl.ds(g * LANES, LANES)] for g in range(GROUPS)]

    def compute(k, xb, ob):
        m = jnp.where(k <= 3, jnp.float32(1.0), jnp.float32(0.0))
        lpk = [lp1[g] + (lp0[g] - lp1[g]) * m for g in range(GROUPS)]
        poff = (k - 4 * jnp.where(k >= 4, 1, 0)) * CH

        @plsc.parallel_loop(0, CH, unroll=1)
        def _(r):
            pi = poff + r
            for g in range(GROUPS):
                col = pl.ds(g * LANES, LANES)
                pv = pb[pi, col] + lpk[g]
                for b in range(BATCH):
                    ob[r, b, col] = xb[b, r, col] + pv

    def start_out(k, ob, sem):
        obase = 1 + s0 + k * CH + jnp.where(k >= 4, SEQ - 4 * CH, 0)
        pltpu.async_copy(ob, out.at[pl.ds(obase, CH)], sem)

    @pl.when(w == 0)
    def _():
        for b in range(BATCH):
            for g in range(GROUPS):
                col = pl.ds(g * LANES, LANES)
                rv[0, b, col] = rot_v[0, col]
        pltpu.async_copy(rv, out.at[pl.ds(0, 1)], s_small)
        pltpu.make_async_copy(rv, out.at[pl.ds(0, 1)], s_small).wait()

    wait_in(xb0, s_in0)
    compute(jnp.int32(0), xb0, ob0)
    start_out(jnp.int32(0), ob0, s_out0)

    start_in(jnp.int32(2), xb0, s_in0)
    wait_in(xb1, s_in1)
    compute(jnp.int32(1), xb1, ob1)
    start_out(jnp.int32(1), ob1, s_out1)

    def round_body(t, acc):
        k1 = 2 * t + 2

        start_in(k1 + 1, xb1, s_in1)
        wait_in(xb0, s_in0)
        wait_out(ob0, s_out0)
        compute(k1, xb0, ob0)
        start_out(k1, ob0, s_out0)

        @pl.when(k1 + 2 < NCHUNK)
        def _():
            start_in(k1 + 2, xb0, s_in0)

        wait_in(xb1, s_in1)
        wait_out(ob1, s_out1)
        compute(k1 + 1, xb1, ob1)
        start_out(k1 + 1, ob1, s_out1)
        return acc

    lax.fori_loop(0, NCHUNK // 2 - 1, round_body, 0)

    wait_out(ob0, s_out0)
    wait_out(ob1, s_out1)


def kernel(x0, x1, unique_pos_w, layer_pos_w, rot_token_w):
    mesh = plsc.VectorSubcoreMesh(core_axis_name="c", subcore_axis_name="s")
    f32 = jnp.float32
    run = pl.kernel(
        _body,
        out_type=jax.ShapeDtypeStruct((NUM_INPUTS * SEQ + 1, BATCH, EMB), f32),
        mesh=mesh,
        scratch_types=[
            pltpu.VMEM((WROWS, EMB), f32),
            pltpu.VMEM((BATCH, CH, EMB), f32),
            pltpu.VMEM((BATCH, CH, EMB), f32),
            pltpu.VMEM((CH, BATCH, EMB), f32),
            pltpu.VMEM((CH, BATCH, EMB), f32),
            pltpu.VMEM((1, BATCH, EMB), f32),
            pltpu.VMEM((NUM_INPUTS, EMB), f32),
            pltpu.VMEM((1, EMB), f32),
            pltpu.SemaphoreType.DMA,
            pltpu.SemaphoreType.DMA,
            pltpu.SemaphoreType.DMA,
            pltpu.SemaphoreType.DMA,
            pltpu.SemaphoreType.DMA,
        ],
    )
    p = run(x0, x1, unique_pos_w, layer_pos_w, rot_token_w)
    return jnp.transpose(p, (1, 0, 2))


# speedup vs baseline: 1.0146x; 1.0025x over previous
import jax
import jax.numpy as jnp
from jax import lax
from jax.experimental import pallas as pl
from jax.experimental.pallas import tpu as pltpu
from jax.experimental.pallas import tpu_sc as plsc

SEQ = 4096
EMB = 128
BATCH = 4
NUM_INPUTS = 2
NUM_CORES = 2
NUM_SUBCORES = 16
NW = NUM_CORES * NUM_SUBCORES
WROWS = SEQ // NW
CH = 32
SUBS = WROWS // CH
NCHUNK = NUM_INPUTS * SUBS
LANES = 16
GROUPS = EMB // LANES


def _body(x0, x1, upw, lpw, rtw, out,
          pb, xb0, xb1, ob0, ob1, rv, lp_v, rot_v,
          s_in0, s_in1, s_out0, s_out1, s_small):
    cid = lax.axis_index("c")
    sid = lax.axis_index("s")
    w = sid * NUM_CORES + cid
    s0 = pl.multiple_of(w * WROWS, WROWS)

    d_lp = pltpu.async_copy(lpw, lp_v, s_small)
    d_rt = pltpu.async_copy(rtw, rot_v, s_small)
    d_pb = pltpu.async_copy(upw.at[pl.ds(s0, WROWS)], pb, s_small)

    def start_in(k, xb, sem):
        @pl.when(k <= SUBS - 1)
        def _():
            ib = pl.multiple_of(s0 + k * CH, CH)
            pltpu.async_copy(x0.at[:, pl.ds(ib, CH)], xb, sem)

        @pl.when(k >= SUBS)
        def _():
            ib = pl.multiple_of(s0 + (k - SUBS) * CH, CH)
            pltpu.async_copy(x1.at[:, pl.ds(ib, CH)], xb, sem)

    def wait_in(xb, sem):
        pltpu.make_async_copy(x0.at[:, pl.ds(0, CH)], xb, sem).wait()

    def wait_out(ob, sem):
        pltpu.make_async_copy(ob, out.at[pl.ds(0, CH)], sem).wait()

    start_in(jnp.int32(0), xb0, s_in0)
    start_in(jnp.int32(1), xb1, s_in1)

    d_lp.wait()
    d_rt.wait()
    d_pb.wait()

    lp0 = [lp_v[0, pl.ds(g * LANES, LANES)] for g in range(GROUPS)]
    lp1 = [lp_v[1, pl.ds(g * LANES, LANES)] for g in range(GROUPS)]

    def compute(k, xb, ob):
        m = jnp.where(k <= SUBS - 1, jnp.float32(1.0), jnp.float32(0.0))
        lpk = [lp1[g] + (lp0[g] - lp1[g]) * m for g in range(GROUPS)]
        poff = (k - SUBS * jnp.where(k >= SUBS, 1, 0)) * CH

        @plsc.parallel_loop(0, CH, unroll=1)
        def _(r):
            pi = poff + r
            for g in range(GROUPS):
                col = pl.ds(g * LANES, LANES)
                pv = pb[pi, col] + lpk[g]
                for b in range(BATCH):
                    ob[r, b, col] = xb[b, r, col] + pv

    def start_out(k, ob, sem):
        obase = 1 + s0 + k * CH + jnp.where(k >= SUBS, SEQ - SUBS * CH, 0)
        pltpu.async_copy(ob, out.at[pl.ds(obase, CH)], sem)

    @pl.when(w == 0)
    def _():
        for b in range(BATCH):
            for g in range(GROUPS):
                col = pl.ds(g * LANES, LANES)
                rv[0, b, col] = rot_v[0, col]
        pltpu.async_copy(rv, out.at[pl.ds(0, 1)], s_small)
        pltpu.make_async_copy(rv, out.at[pl.ds(0, 1)], s_small).wait()

    wait_in(xb0, s_in0)
    compute(jnp.int32(0), xb0, ob0)
    start_out(jnp.int32(0), ob0, s_out0)

    start_in(jnp.int32(2), xb0, s_in0)
    wait_in(xb1, s_in1)
    compute(jnp.int32(1), xb1, ob1)
    start_out(jnp.int32(1), ob1, s_out1)

    def round_body(t, acc):
        k1 = 2 * t + 2

        start_in(k1 + 1, xb1, s_in1)
        wait_in(xb0, s_in0)
        wait_out(ob0, s_out0)
        compute(k1, xb0, ob0)
        start_out(k1, ob0, s_out0)

        @pl.when(k1 + 2 < NCHUNK)
        def _():
            start_in(k1 + 2, xb0, s_in0)

        wait_in(xb1, s_in1)
        wait_out(ob1, s_out1)
        compute(k1 + 1, xb1, ob1)
        start_out(k1 + 1, ob1, s_out1)
        return acc

    lax.fori_loop(0, NCHUNK // 2 - 1, round_body, 0)

    wait_out(ob0, s_out0)
    wait_out(ob1, s_out1)


def kernel(x0, x1, unique_pos_w, layer_pos_w, rot_token_w):
    mesh = plsc.VectorSubcoreMesh(core_axis_name="c", subcore_axis_name="s")
    f32 = jnp.float32
    run = pl.kernel(
        _body,
        out_type=jax.ShapeDtypeStruct((NUM_INPUTS * SEQ + 1, BATCH, EMB), f32),
        mesh=mesh,
        scratch_types=[
            pltpu.VMEM((WROWS, EMB), f32),
            pltpu.VMEM((BATCH, CH, EMB), f32),
            pltpu.VMEM((BATCH, CH, EMB), f32),
            pltpu.VMEM((CH, BATCH, EMB), f32),
            pltpu.VMEM((CH, BATCH, EMB), f32),
            pltpu.VMEM((1, BATCH, EMB), f32),
            pltpu.VMEM((NUM_INPUTS, EMB), f32),
            pltpu.VMEM((1, EMB), f32),
            pltpu.SemaphoreType.DMA,
            pltpu.SemaphoreType.DMA,
            pltpu.SemaphoreType.DMA,
            pltpu.SemaphoreType.DMA,
            pltpu.SemaphoreType.DMA,
        ],
    )
    p = run(x0, x1, unique_pos_w, layer_pos_w, rot_token_w)
    return jnp.transpose(p, (1, 0, 2))
